# Initial kernel scaffold; baseline (speedup 1.0000x reference)
import functools

import jax
import jax.numpy as jnp
from jax import lax
from jax.experimental import pallas as pl
from jax.experimental.pallas import tpu as pltpu
from jax.experimental.pallas import tpu_sc as plsc

N = 10000
IN = 128
HID = 16
HEADS = 8
OUT = 128
E = 320000

NP = 10240
NC = 2
NS = 16
NW = NC * NS
CHUNK = 128
STEPS = 81
EP = NW * CHUNK * STEPS
RPS = NP // NS


def _edge_pass(src, dst, h, a_tab, b_tab, z128, z16):
    mesh = plsc.VectorSubcoreMesh(core_axis_name="c", subcore_axis_name="s")

    @functools.partial(
        pl.kernel,
        out_type=[
            jax.ShapeDtypeStruct((NC, NP, 128), jnp.float32),
            jax.ShapeDtypeStruct((NC, NP, 16), jnp.float32),
        ],
        mesh=mesh,
        scratch_types=[
            pltpu.VMEM((CHUNK,), jnp.int32),
            pltpu.VMEM((CHUNK,), jnp.int32),
            pltpu.VMEM((CHUNK, 128), jnp.float32),
            pltpu.VMEM((CHUNK, 16), jnp.float32),
            pltpu.VMEM((CHUNK, 16), jnp.float32),
            pltpu.VMEM((CHUNK, 16), jnp.float32),
            pltpu.VMEM_SHARED((NP, 128), jnp.float32),
            pltpu.VMEM_SHARED((NP, 16), jnp.float32),
            pltpu.SemaphoreType.DMA,
            pltpu.SemaphoreType.DMA,
            pltpu.SemaphoreType.DMA,
        ],
    )
    def k(src_h, dst_h, h_h, a_h, b_h, z128_h, z16_h, acc_o, den_o,
          src_v, dst_v, rows_v, a_v, b_v, w_v, acc_sh, den_sh, s0, s1, s2):
        cid = lax.axis_index("c")
        sid = lax.axis_index("s")
        wid = cid * NS + sid
        r0 = sid * RPS
        pltpu.sync_copy(z128_h.at[pl.ds(r0, RPS)], acc_sh.at[pl.ds(r0, RPS)])
        pltpu.sync_copy(z16_h.at[pl.ds(r0, RPS)], den_sh.at[pl.ds(r0, RPS)])
        plsc.subcore_barrier()

        base = wid * STEPS * CHUNK

        @pl.loop(0, STEPS)
        def _(st):
            off = base + st * CHUNK
            pltpu.sync_copy(src_h.at[pl.ds(off, CHUNK)], src_v)
            pltpu.sync_copy(dst_h.at[pl.ds(off, CHUNK)], dst_v)
            ca = pltpu.async_copy(a_h.at[src_v], a_v, s0)
            cb = pltpu.async_copy(b_h.at[dst_v], b_v, s1)
            cr = pltpu.async_copy(h_h.at[src_v], rows_v, s2)
            ca.wait()
            cb.wait()

            @pl.loop(0, CHUNK)
            def _(e):
                t = a_v[e, :] + b_v[e, :]
                t = jnp.where(t >= 0.0, t, 0.2 * t)
                w_v[e, :] = jnp.exp(t)

            pltpu.sync_copy(w_v, den_sh.at[dst_v], add=True)
            cr.wait()

            @pl.loop(0, CHUNK)
            def _(e):
                for m in range(8):
                    ws = w_v[e, m]
                    rows_v[e, pl.ds(16 * m, 16)] = (
                        rows_v[e, pl.ds(16 * m, 16)] * ws)

            pltpu.sync_copy(rows_v, acc_sh.at[dst_v], add=True)

        plsc.subcore_barrier()
        pltpu.sync_copy(acc_sh.at[pl.ds(r0, RPS)], acc_o.at[cid, pl.ds(r0, RPS)])
        pltpu.sync_copy(den_sh.at[pl.ds(r0, RPS)], den_o.at[cid, pl.ds(r0, RPS)])

    return k(src, dst, h, a_tab, b_tab, z128, z16)


_BM = 1024


def _dense1(x_p, w1, ms, md):
    def body(x_ref, w_ref, ms_ref, md_ref, h_ref, a_ref, b_ref):
        h = jnp.dot(x_ref[...], w_ref[...], preferred_element_type=jnp.float32)
        h_ref[...] = h
        a_ref[...] = jnp.dot(h, ms_ref[...], preferred_element_type=jnp.float32)
        b_ref[...] = jnp.dot(h, md_ref[...], preferred_element_type=jnp.float32)

    return pl.pallas_call(
        body,
        grid=(NP // _BM,),
        in_specs=[
            pl.BlockSpec((_BM, 128), lambda i: (i, 0)),
            pl.BlockSpec((128, 128), lambda i: (0, 0)),
            pl.BlockSpec((128, 16), lambda i: (0, 0)),
            pl.BlockSpec((128, 16), lambda i: (0, 0)),
        ],
        out_specs=[
            pl.BlockSpec((_BM, 128), lambda i: (i, 0)),
            pl.BlockSpec((_BM, 16), lambda i: (i, 0)),
            pl.BlockSpec((_BM, 16), lambda i: (i, 0)),
        ],
        out_shape=[
            jax.ShapeDtypeStruct((NP, 128), jnp.float32),
            jax.ShapeDtypeStruct((NP, 16), jnp.float32),
            jax.ShapeDtypeStruct((NP, 16), jnp.float32),
        ],
    )(x_p, w1, ms, md)


def _dense2(p0, p1, d0, d1, r_sel, b1, w2, ms, md):
    def body(p0_ref, p1_ref, d0_ref, d1_ref, r_ref, b1_ref, w2_ref,
             ms_ref, md_ref, h_ref, a_ref, b_ref):
        den = jnp.dot(d0_ref[...] + d1_ref[...], r_ref[...],
                      preferred_element_type=jnp.float32)
        agg = (p0_ref[...] + p1_ref[...]) / jnp.maximum(den, 1e-16)
        v = agg + b1_ref[...]
        x2 = jnp.where(v > 0.0, v, jnp.expm1(v))
        h2 = jnp.dot(x2, w2_ref[...], preferred_element_type=jnp.float32)
        h_ref[...] = h2
        a_ref[...] = jnp.dot(h2, ms_ref[...], preferred_element_type=jnp.float32)
        b_ref[...] = jnp.dot(h2, md_ref[...], preferred_element_type=jnp.float32)

    return pl.pallas_call(
        body,
        grid=(NP // _BM,),
        in_specs=[
            pl.BlockSpec((_BM, 128), lambda i: (i, 0)),
            pl.BlockSpec((_BM, 128), lambda i: (i, 0)),
            pl.BlockSpec((_BM, 16), lambda i: (i, 0)),
            pl.BlockSpec((_BM, 16), lambda i: (i, 0)),
            pl.BlockSpec((16, 128), lambda i: (0, 0)),
            pl.BlockSpec((1, 128), lambda i: (0, 0)),
            pl.BlockSpec((128, 128), lambda i: (0, 0)),
            pl.BlockSpec((128, 16), lambda i: (0, 0)),
            pl.BlockSpec((128, 16), lambda i: (0, 0)),
        ],
        out_specs=[
            pl.BlockSpec((_BM, 128), lambda i: (i, 0)),
            pl.BlockSpec((_BM, 16), lambda i: (i, 0)),
            pl.BlockSpec((_BM, 16), lambda i: (i, 0)),
        ],
        out_shape=[
            jax.ShapeDtypeStruct((NP, 128), jnp.float32),
            jax.ShapeDtypeStruct((NP, 16), jnp.float32),
            jax.ShapeDtypeStruct((NP, 16), jnp.float32),
        ],
    )(p0, p1, d0, d1, r_sel, b1, w2, ms, md)


_BMC = 2000


def _dense3(p0, p1, d0, d1, r_sel, b2):
    def body(p0_ref, p1_ref, d0_ref, d1_ref, r_ref, b2_ref, o_ref):
        den = jnp.dot(d0_ref[...] + d1_ref[...], r_ref[...],
                      preferred_element_type=jnp.float32)
        o_ref[...] = ((p0_ref[...] + p1_ref[...])
                      / jnp.maximum(den, 1e-16) + b2_ref[...])

    return pl.pallas_call(
        body,
        grid=(N // _BMC,),
        in_specs=[
            pl.BlockSpec((_BMC, 128), lambda i: (i, 0)),
            pl.BlockSpec((_BMC, 128), lambda i: (i, 0)),
            pl.BlockSpec((_BMC, 16), lambda i: (i, 0)),
            pl.BlockSpec((_BMC, 16), lambda i: (i, 0)),
            pl.BlockSpec((16, 128), lambda i: (0, 0)),
            pl.BlockSpec((1, 128), lambda i: (0, 0)),
        ],
        out_specs=pl.BlockSpec((_BMC, 128), lambda i: (i, 0)),
        out_shape=jax.ShapeDtypeStruct((N, 128), jnp.float32),
    )(p0, p1, d0, d1, r_sel, b2)


def kernel(x, edge_index, W1, att_src1, att_dst1, b1, W2, att_src2, att_dst2, b2):
    f32 = jnp.float32
    loops = jnp.arange(N, dtype=jnp.int32)
    src = jnp.concatenate([edge_index[0], loops])
    dst = jnp.concatenate([edge_index[1], loops])
    pad = EP - (E + N)
    padv = jnp.full((pad,), N, jnp.int32)
    src_p = jnp.concatenate([src, padv])
    dst_p = jnp.concatenate([dst, padv])

    x_p = jnp.zeros((NP, IN), f32).at[:N].set(x)

    eye8 = jnp.eye(HEADS, dtype=f32)
    ms1 = (att_src1[:, :, None] * eye8[:, None, :]).reshape(HEADS * HID, HEADS)
    md1 = (att_dst1[:, :, None] * eye8[:, None, :]).reshape(HEADS * HID, HEADS)
    ms1 = jnp.concatenate([ms1, ms1], axis=1)
    md1 = jnp.concatenate([md1, md1], axis=1)
    ms2 = jnp.tile(att_src2.T, (1, 16))
    md2 = jnp.tile(att_dst2.T, (1, 16))

    r1 = (jnp.arange(128)[None, :] // 16 == jnp.arange(16)[:, None]).astype(f32)
    r2 = (jnp.arange(16)[:, None] == 0).astype(f32) * jnp.ones((1, 128), f32)

    z128 = jnp.zeros((NP, 128), f32)
    z16 = jnp.zeros((NP, 16), f32)

    h1, a1, btab1 = _dense1(x_p, W1, ms1, md1)
    acc1, den1 = _edge_pass(src_p, dst_p, h1, a1, btab1, z128, z16)
    h2, a2, btab2 = _dense2(acc1[0], acc1[1], den1[0], den1[1], r1,
                            b1.reshape(1, 128), W2, ms2, md2)
    acc2, den2 = _edge_pass(src_p, dst_p, h2, a2, btab2, z128, z16)
    out = _dense3(acc2[0], acc2[1], den2[0], den2[1], r2, b2.reshape(1, 128))
    return out


# speedup vs baseline: 30.2635x; 30.2635x over previous
import functools

import jax
import jax.numpy as jnp
from jax import lax
from jax.experimental import pallas as pl
from jax.experimental.pallas import tpu as pltpu
from jax.experimental.pallas import tpu_sc as plsc

N = 10000
IN = 128
HID = 16
HEADS = 8
OUT = 128
E = 320000

NP = 10240
NC = 2
NS = 16
NW = NC * NS
CHUNK = 48
STEPS = 215
EP = NW * CHUNK * STEPS
RPS = NP // NS
RPSQ = NP // 8 // NS


def _edge_pass(src, dst, h, a_tab, b_tab):
    mesh = plsc.VectorSubcoreMesh(core_axis_name="c", subcore_axis_name="s",
                                  num_cores=NC, num_subcores=NS)

    @functools.partial(
        pl.kernel,
        out_type=[
            jax.ShapeDtypeStruct((NC, NP, 128), jnp.float32),
            jax.ShapeDtypeStruct((NC, NP // 8, 128), jnp.float32),
        ],
        mesh=mesh,
        scratch_types=[
            pltpu.VMEM((CHUNK,), jnp.int32),
            pltpu.VMEM((CHUNK,), jnp.int32),
            pltpu.VMEM((CHUNK, 128), jnp.float32),
            pltpu.VMEM((CHUNK, 128), jnp.float32),
            pltpu.VMEM((CHUNK, 128), jnp.float32),
            pltpu.VMEM((CHUNK, 128), jnp.float32),
            pltpu.VMEM((CHUNK,), jnp.int32),
            pltpu.VMEM((CHUNK, 128), jnp.float32),
            pltpu.VMEM_SHARED((NP // 8, 128), jnp.float32),
            pltpu.VMEM_SHARED((NP, 128), jnp.float32),
            pltpu.SemaphoreType.DMA,
            pltpu.SemaphoreType.DMA,
            pltpu.SemaphoreType.DMA,
        ],
    )
    def k(src_h, dst_h, h_h, a_h, b_h, acc_o, den_o,
          src_v, dst_v, rows_v, a_v, b_v, w_v, dstq_v, wz_v, den_sh, acc_sh,
          s0, s1, s2):
        cid = lax.axis_index("c")
        sid = lax.axis_index("s")
        wid = cid * NS + sid
        r0 = sid * RPS
        @pl.loop(0, CHUNK)
        def _(e):
            for c in range(8):
                wz_v[e, pl.ds(16 * c, 16)] = jnp.zeros((16,), jnp.float32)

        for p in range(RPS // 40):
            pltpu.sync_copy(wz_v.at[pl.ds(0, 40)],
                            acc_sh.at[pl.ds(r0 + p * 40, 40)])
        for q in range(2):
            pltpu.sync_copy(wz_v.at[pl.ds(0, RPSQ // 2)],
                            den_sh.at[pl.ds(sid * RPSQ + q * (RPSQ // 2),
                                            RPSQ // 2)])
        plsc.subcore_barrier()

        base = wid * STEPS * CHUNK

        @pl.loop(0, STEPS)
        def _(st):
            off = base + st * CHUNK
            pltpu.sync_copy(src_h.at[pl.ds(off, CHUNK)], src_v)
            pltpu.sync_copy(dst_h.at[pl.ds(off, CHUNK)], dst_v)
            ca = pltpu.async_copy(a_h.at[src_v], a_v, s0)
            cb = pltpu.async_copy(b_h.at[dst_v], b_v, s1)
            cr = pltpu.async_copy(h_h.at[src_v], rows_v, s2)
            ca.wait()
            cb.wait()

            @pl.loop(0, CHUNK // 16)
            def _(j):
                d16 = dst_v[pl.ds(j * 16, 16)]
                dstq_v[pl.ds(j * 16, 16)] = lax.shift_right_logical(d16, 3)
                slot16 = (d16 & 7) * 16
                for i in range(16):
                    e = j * 16 + i
                    t = a_v[e, pl.ds(0, 16)] + b_v[e, pl.ds(0, 16)]
                    t = jnp.where(t >= 0.0, t, 0.2 * t)
                    w = jnp.exp(t)
                    w_v[e, pl.ds(0, 16)] = w
                    wz_v[e, pl.ds(slot16[i], 16)] = w

            pltpu.sync_copy(wz_v, den_sh.at[dstq_v], add=True)

            @pl.loop(0, CHUNK // 16)
            def _(j):
                d16 = dst_v[pl.ds(j * 16, 16)]
                slot16 = (d16 & 7) * 16
                for i in range(16):
                    wz_v[j * 16 + i, pl.ds(slot16[i], 16)] = (
                        jnp.zeros((16,), jnp.float32))

            cr.wait()

            @pl.loop(0, CHUNK)
            def _(e):
                wrow = w_v[e, pl.ds(0, 16)]
                for m in range(8):
                    ws = wrow[m]
                    rows_v[e, pl.ds(16 * m, 16)] = (
                        rows_v[e, pl.ds(16 * m, 16)] * ws)

            pltpu.sync_copy(rows_v, acc_sh.at[dst_v], add=True)

        plsc.subcore_barrier()
        pltpu.sync_copy(acc_sh.at[pl.ds(r0, RPS)], acc_o.at[cid, pl.ds(r0, RPS)])
        pltpu.sync_copy(den_sh.at[pl.ds(sid * RPSQ, RPSQ)],
                        den_o.at[cid, pl.ds(sid * RPSQ, RPSQ)])

    return k(src, dst, h, a_tab, b_tab)


_BM = 1024


def _dense1(x_p, w1, ms, md):
    def body(x_ref, w_ref, ms_ref, md_ref, h_ref, a_ref, b_ref):
        h = jnp.dot(x_ref[...], w_ref[...], preferred_element_type=jnp.float32)
        h_ref[...] = h
        a_ref[...] = jnp.dot(h, ms_ref[...], preferred_element_type=jnp.float32)
        b_ref[...] = jnp.dot(h, md_ref[...], preferred_element_type=jnp.float32)

    return pl.pallas_call(
        body,
        grid=(NP // _BM,),
        in_specs=[
            pl.BlockSpec((_BM, 128), lambda i: (i, 0)),
            pl.BlockSpec((128, 128), lambda i: (0, 0)),
            pl.BlockSpec((128, 128), lambda i: (0, 0)),
            pl.BlockSpec((128, 128), lambda i: (0, 0)),
        ],
        out_specs=[
            pl.BlockSpec((_BM, 128), lambda i: (i, 0)),
            pl.BlockSpec((_BM, 128), lambda i: (i, 0)),
            pl.BlockSpec((_BM, 128), lambda i: (i, 0)),
        ],
        out_shape=[
            jax.ShapeDtypeStruct((NP, 128), jnp.float32),
            jax.ShapeDtypeStruct((NP, 128), jnp.float32),
            jax.ShapeDtypeStruct((NP, 128), jnp.float32),
        ],
    )(x_p, w1, ms, md)


def _dense2(p0, p1, d0, d1, r_sel, b1, w2, ms, md):
    def body(p0_ref, p1_ref, d0_ref, d1_ref, r_ref, b1_ref, w2_ref,
             ms_ref, md_ref, h_ref, a_ref, b_ref):
        den = jnp.dot(d0_ref[...] + d1_ref[...], r_ref[...],
                      preferred_element_type=jnp.float32)
        agg = (p0_ref[...] + p1_ref[...]) / jnp.maximum(den, 1e-16)
        v = agg + b1_ref[...]
        x2 = jnp.where(v > 0.0, v, jnp.exp(v) - 1.0)
        h2 = jnp.dot(x2, w2_ref[...], preferred_element_type=jnp.float32)
        h_ref[...] = h2
        a_ref[...] = jnp.dot(h2, ms_ref[...], preferred_element_type=jnp.float32)
        b_ref[...] = jnp.dot(h2, md_ref[...], preferred_element_type=jnp.float32)

    return pl.pallas_call(
        body,
        grid=(NP // _BM,),
        in_specs=[
            pl.BlockSpec((_BM, 128), lambda i: (i, 0)),
            pl.BlockSpec((_BM, 128), lambda i: (i, 0)),
            pl.BlockSpec((_BM, 16), lambda i: (i, 0)),
            pl.BlockSpec((_BM, 16), lambda i: (i, 0)),
            pl.BlockSpec((16, 128), lambda i: (0, 0)),
            pl.BlockSpec((1, 128), lambda i: (0, 0)),
            pl.BlockSpec((128, 128), lambda i: (0, 0)),
            pl.BlockSpec((128, 128), lambda i: (0, 0)),
            pl.BlockSpec((128, 128), lambda i: (0, 0)),
        ],
        out_specs=[
            pl.BlockSpec((_BM, 128), lambda i: (i, 0)),
            pl.BlockSpec((_BM, 128), lambda i: (i, 0)),
            pl.BlockSpec((_BM, 128), lambda i: (i, 0)),
        ],
        out_shape=[
            jax.ShapeDtypeStruct((NP, 128), jnp.float32),
            jax.ShapeDtypeStruct((NP, 128), jnp.float32),
            jax.ShapeDtypeStruct((NP, 128), jnp.float32),
        ],
    )(p0, p1, d0, d1, r_sel, b1, w2, ms, md)


_BMC = 2000


def _dense3(p0, p1, d0, d1, r_sel, b2):
    def body(p0_ref, p1_ref, d0_ref, d1_ref, r_ref, b2_ref, o_ref):
        den = jnp.dot(d0_ref[...] + d1_ref[...], r_ref[...],
                      preferred_element_type=jnp.float32)
        o_ref[...] = ((p0_ref[...] + p1_ref[...])
                      / jnp.maximum(den, 1e-16) + b2_ref[...])

    return pl.pallas_call(
        body,
        grid=(N // _BMC,),
        in_specs=[
            pl.BlockSpec((_BMC, 128), lambda i: (i, 0)),
            pl.BlockSpec((_BMC, 128), lambda i: (i, 0)),
            pl.BlockSpec((_BMC, 16), lambda i: (i, 0)),
            pl.BlockSpec((_BMC, 16), lambda i: (i, 0)),
            pl.BlockSpec((16, 128), lambda i: (0, 0)),
            pl.BlockSpec((1, 128), lambda i: (0, 0)),
        ],
        out_specs=pl.BlockSpec((_BMC, 128), lambda i: (i, 0)),
        out_shape=jax.ShapeDtypeStruct((N, 128), jnp.float32),
    )(p0, p1, d0, d1, r_sel, b2)


def kernel(x, edge_index, W1, att_src1, att_dst1, b1, W2, att_src2, att_dst2, b2):
    f32 = jnp.float32
    loops = jnp.arange(N, dtype=jnp.int32)
    src = jnp.concatenate([edge_index[0], loops])
    dst = jnp.concatenate([edge_index[1], loops])
    pad = EP - (E + N)
    padv = jnp.full((pad,), N, jnp.int32)
    src_p = jnp.concatenate([src, padv])
    dst_p = jnp.concatenate([dst, padv])

    x_p = jnp.zeros((NP, IN), f32).at[:N].set(x)

    eye8 = jnp.eye(HEADS, dtype=f32)
    ms1 = (att_src1[:, :, None] * eye8[:, None, :]).reshape(HEADS * HID, HEADS)
    md1 = (att_dst1[:, :, None] * eye8[:, None, :]).reshape(HEADS * HID, HEADS)
    zpad = jnp.zeros((HEADS * HID, 112), f32)
    ms1 = jnp.concatenate([ms1, ms1, zpad], axis=1)
    md1 = jnp.concatenate([md1, md1, zpad], axis=1)
    ms2 = jnp.concatenate([jnp.tile(att_src2.T, (1, 16)), zpad], axis=1)
    md2 = jnp.concatenate([jnp.tile(att_dst2.T, (1, 16)), zpad], axis=1)

    r1 = (jnp.arange(128)[None, :] // 16 == jnp.arange(16)[:, None]).astype(f32)
    r2 = (jnp.arange(16)[:, None] == 0).astype(f32) * jnp.ones((1, 128), f32)

    h1, a1, btab1 = _dense1(x_p, W1, ms1, md1)
    acc1, den1 = _edge_pass(src_p, dst_p, h1, a1, btab1)
    den1 = den1.reshape(NC, NP, 16)
    h2, a2, btab2 = _dense2(acc1[0], acc1[1], den1[0], den1[1], r1,
                            b1.reshape(1, 128), W2, ms2, md2)
    acc2, den2 = _edge_pass(src_p, dst_p, h2, a2, btab2)
    den2 = den2.reshape(NC, NP, 16)
    out = _dense3(acc2[0], acc2[1], den2[0], den2[1], r2, b2.reshape(1, 128))
    return out
